# baseline (speedup 1.0000x reference)
import jax
import jax.numpy as jnp
from jax import lax
from jax.experimental import pallas as pl

VOCAB = 1000
D = 128
B, S = 1024, 20
BLK_B = 64
NB = B // BLK_B
TPB = BLK_B * S
SHIFT_PAD = 32


def _precompute_body(char_embed_ref, w1c_ref, shift_pad_ref, w1s_ref, b1_ref,
                     a_ref, c_ref):
    a_ref[...] = jnp.dot(char_embed_ref[...], w1c_ref[...],
                         preferred_element_type=jnp.float32
                         ).astype(jnp.bfloat16)
    c_ref[...] = (jnp.dot(shift_pad_ref[...], w1s_ref[...],
                          preferred_element_type=jnp.float32)
                  + b1_ref[...]).astype(jnp.bfloat16)


def _main_body(char_ids_ref, shift_ids_ref, a_ref, c_ref, w2_ref, b2_ref,
               out_ref):
    ids = char_ids_ref[0]
    oh_c = (ids == lax.broadcasted_iota(jnp.int32, (TPB, VOCAB), 1)
            ).astype(jnp.bfloat16)
    g = jnp.dot(oh_c, a_ref[...], preferred_element_type=jnp.float32)
    sid = shift_ids_ref[0]
    oh_s = (sid == lax.broadcasted_iota(jnp.int32, (TPB, SHIFT_PAD), 1)
            ).astype(jnp.bfloat16)
    g = g + jnp.dot(oh_s, c_ref[...], preferred_element_type=jnp.float32)
    h = jnp.maximum(g, 0.0).astype(jnp.bfloat16)
    for s in range(S):
        h_s = h[s * BLK_B:(s + 1) * BLK_B, :]
        out_ref[:, s, :] = jnp.dot(
            h_s, w2_ref[...], preferred_element_type=jnp.float32) + b2_ref[...]


def kernel(x_chars, x_shifts, char_embed, shift_embed, W1, b1, W2, b2):
    x_chars = x_chars.astype(jnp.int32)
    x_shifts = x_shifts.astype(jnp.int32)
    w1c = W1[:D, :]
    w1s = W1[D:, :]
    shift_pad = jnp.zeros((SHIFT_PAD, D), jnp.float32).at[:26, :].set(shift_embed)

    a_tab, c_tab = pl.pallas_call(
        _precompute_body,
        out_shape=(
            jax.ShapeDtypeStruct((VOCAB, D), jnp.bfloat16),
            jax.ShapeDtypeStruct((SHIFT_PAD, D), jnp.bfloat16),
        ),
    )(char_embed, w1c, shift_pad, w1s, b1.reshape(1, D))

    char_ids = jnp.transpose(x_chars.reshape(NB, BLK_B, S), (0, 2, 1)
                             ).reshape(NB, TPB, 1)
    shift_ids = jnp.broadcast_to(x_shifts.reshape(NB, 1, BLK_B),
                                 (NB, S, BLK_B)).reshape(NB, TPB, 1)

    out = pl.pallas_call(
        _main_body,
        grid=(NB,),
        in_specs=[
            pl.BlockSpec((1, TPB, 1), lambda i: (i, 0, 0)),
            pl.BlockSpec((1, TPB, 1), lambda i: (i, 0, 0)),
            pl.BlockSpec((VOCAB, D), lambda i: (0, 0)),
            pl.BlockSpec((SHIFT_PAD, D), lambda i: (0, 0)),
            pl.BlockSpec((D, VOCAB), lambda i: (0, 0)),
            pl.BlockSpec((1, VOCAB), lambda i: (0, 0)),
        ],
        out_specs=pl.BlockSpec((BLK_B, S, VOCAB), lambda i: (i, 0, 0)),
        out_shape=jax.ShapeDtypeStruct((B, S, VOCAB), jnp.float32),
    )(char_ids, shift_ids, a_tab, c_tab, W2.astype(jnp.bfloat16),
      b2.reshape(1, VOCAB))

    return out


# speedup vs baseline: 1.0764x; 1.0764x over previous
import jax
import jax.numpy as jnp
from jax import lax
from jax.experimental import pallas as pl
from jax.experimental.pallas import tpu as pltpu

VOCAB = 1000
D = 128
B, S = 1024, 20
BLK_B = 128
NB = B // BLK_B
TPB = BLK_B * S
SHIFT_PAD = 32


def _precompute_body(char_embed_ref, w1c_ref, shift_pad_ref, w1s_ref, b1_ref,
                     a_ref, c_ref):
    a_ref[...] = jnp.dot(char_embed_ref[...], w1c_ref[...],
                         preferred_element_type=jnp.float32
                         ).astype(jnp.bfloat16)
    c_ref[...] = (jnp.dot(shift_pad_ref[...], w1s_ref[...],
                          preferred_element_type=jnp.float32)
                  + b1_ref[...]).astype(jnp.bfloat16)


def _main_body(char_ids_ref, shift_ids_ref, a_ref, c_ref, w2_ref, b2_ref,
               out_ref):
    ids = char_ids_ref[0]
    oh_c = (ids == lax.broadcasted_iota(jnp.int32, (TPB, VOCAB), 1)
            ).astype(jnp.bfloat16)
    g = jnp.dot(oh_c, a_ref[...], preferred_element_type=jnp.float32)
    sid = shift_ids_ref[0]
    oh_s = (sid == lax.broadcasted_iota(jnp.int32, (TPB, SHIFT_PAD), 1)
            ).astype(jnp.bfloat16)
    g = g + jnp.dot(oh_s, c_ref[...], preferred_element_type=jnp.float32)
    h = jnp.maximum(g, 0.0).astype(jnp.bfloat16)
    for s in range(S):
        h_s = h[s * BLK_B:(s + 1) * BLK_B, :]
        out_ref[:, s, :] = jnp.dot(
            h_s, w2_ref[...], preferred_element_type=jnp.float32) + b2_ref[...]


def kernel(x_chars, x_shifts, char_embed, shift_embed, W1, b1, W2, b2):
    x_chars = x_chars.astype(jnp.int32)
    x_shifts = x_shifts.astype(jnp.int32)
    w1c = W1[:D, :]
    w1s = W1[D:, :]
    shift_pad = jnp.zeros((SHIFT_PAD, D), jnp.float32).at[:26, :].set(shift_embed)

    a_tab, c_tab = pl.pallas_call(
        _precompute_body,
        out_shape=(
            jax.ShapeDtypeStruct((VOCAB, D), jnp.bfloat16),
            jax.ShapeDtypeStruct((SHIFT_PAD, D), jnp.bfloat16),
        ),
    )(char_embed, w1c, shift_pad, w1s, b1.reshape(1, D))

    char_ids = jnp.transpose(x_chars.reshape(NB, BLK_B, S), (0, 2, 1)
                             ).reshape(NB, TPB, 1)
    shift_ids = jnp.broadcast_to(x_shifts.reshape(NB, 1, BLK_B),
                                 (NB, S, BLK_B)).reshape(NB, TPB, 1)

    out = pl.pallas_call(
        _main_body,
        grid=(NB,),
        in_specs=[
            pl.BlockSpec((1, TPB, 1), lambda i: (i, 0, 0)),
            pl.BlockSpec((1, TPB, 1), lambda i: (i, 0, 0)),
            pl.BlockSpec((VOCAB, D), lambda i: (0, 0)),
            pl.BlockSpec((SHIFT_PAD, D), lambda i: (0, 0)),
            pl.BlockSpec((D, VOCAB), lambda i: (0, 0)),
            pl.BlockSpec((1, VOCAB), lambda i: (0, 0)),
        ],
        out_specs=pl.BlockSpec((BLK_B, S, VOCAB), lambda i: (i, 0, 0)),
        out_shape=jax.ShapeDtypeStruct((B, S, VOCAB), jnp.float32),
        compiler_params=pltpu.CompilerParams(
            dimension_semantics=("parallel",)),
    )(char_ids, shift_ids, a_tab, c_tab, W2.astype(jnp.bfloat16),
      b2.reshape(1, VOCAB))

    return out


# speedup vs baseline: 3.7194x; 3.4554x over previous
import jax
import jax.numpy as jnp
from jax import lax
from jax.experimental import pallas as pl

VOCAB = 1000
D = 128
B, S = 1024, 20
SHIFT_PAD = 32


def _precompute_body(w1ct_ref, char_t_ref, w1st_ref, shift_t_ref, b1_ref,
                     shifts_ref, at_ref, shiftT_ref):
    at_ref[...] = jnp.dot(w1ct_ref[...], char_t_ref[...],
                          preferred_element_type=jnp.float32
                          ).astype(jnp.bfloat16)
    ct = jnp.dot(w1st_ref[...], shift_t_ref[...],
                 preferred_element_type=jnp.float32)
    ohs = (shifts_ref[...] ==
           lax.broadcasted_iota(jnp.int32, (SHIFT_PAD, B), 0)
           ).astype(jnp.float32)
    shiftT_ref[...] = jnp.dot(ct, ohs,
                              preferred_element_type=jnp.float32) + b1_ref[...]


def _main_body(ids_ref, at_ref, shiftT_ref, w2t_ref, b2_ref, out_ref):
    ids = ids_ref[0]
    oh = (ids == lax.broadcasted_iota(jnp.int32, (VOCAB, B), 0)
          ).astype(jnp.bfloat16)
    g = jnp.dot(at_ref[...], oh,
                preferred_element_type=jnp.float32)
    h = jnp.maximum(g + shiftT_ref[...], 0.0).astype(jnp.bfloat16)
    out_ref[0] = jnp.dot(w2t_ref[...], h,
                         preferred_element_type=jnp.float32) + b2_ref[...]


def kernel(x_chars, x_shifts, char_embed, shift_embed, W1, b1, W2, b2):
    x_chars = x_chars.astype(jnp.int32)
    x_shifts = x_shifts.astype(jnp.int32)
    w1ct = W1[:D, :].T
    w1st = W1[D:, :].T
    char_t = char_embed.T
    shift_t = jnp.zeros((D, SHIFT_PAD), jnp.float32).at[:, :26].set(
        shift_embed.T)

    at_tab, shiftT = pl.pallas_call(
        _precompute_body,
        out_shape=(
            jax.ShapeDtypeStruct((D, VOCAB), jnp.bfloat16),
            jax.ShapeDtypeStruct((D, B), jnp.float32),
        ),
    )(w1ct, char_t, w1st, shift_t, b1.reshape(D, 1), x_shifts.reshape(1, B))

    ids_t = x_chars.T.reshape(S, 1, B)

    res = pl.pallas_call(
        _main_body,
        grid=(S,),
        in_specs=[
            pl.BlockSpec((1, 1, B), lambda s: (s, 0, 0)),
            pl.BlockSpec((D, VOCAB), lambda s: (0, 0)),
            pl.BlockSpec((D, B), lambda s: (0, 0)),
            pl.BlockSpec((VOCAB, D), lambda s: (0, 0)),
            pl.BlockSpec((VOCAB, 1), lambda s: (0, 0)),
        ],
        out_specs=pl.BlockSpec((1, VOCAB, B), lambda s: (s, 0, 0)),
        out_shape=jax.ShapeDtypeStruct((S, VOCAB, B), jnp.float32),
    )(ids_t, at_tab, shiftT, W2.T.astype(jnp.bfloat16), b2.reshape(VOCAB, 1))

    return jnp.transpose(res, (2, 0, 1))


# speedup vs baseline: 4.4170x; 1.1876x over previous
import jax
import jax.numpy as jnp
from jax import lax
from jax.experimental import pallas as pl

VOCAB = 1000
D = 128
B, S = 1024, 20
NSHIFT = 26


def _precompute_body(w1b_ref, char_ref, shift_ref, shifts_ref,
                     at_ref, shiftT_ref):
    w1c = w1b_ref[0:D]
    at_ref[...] = lax.dot_general(
        w1c, char_ref[...], (((0,), (1,)), ((), ())),
        preferred_element_type=jnp.float32).astype(jnp.bfloat16)
    w1s_aug = w1b_ref[D:]
    se_aug = jnp.concatenate(
        [shift_ref[...], jnp.ones((NSHIFT, 1), jnp.float32)], axis=1)
    ct = lax.dot_general(
        w1s_aug, se_aug, (((0,), (1,)), ((), ())),
        preferred_element_type=jnp.float32)
    ohs = (shifts_ref[...] ==
           lax.broadcasted_iota(jnp.int32, (NSHIFT, B), 0)
           ).astype(jnp.float32)
    shiftT_ref[...] = jnp.dot(ct, ohs, preferred_element_type=jnp.float32)


def _main_body(ids_ref, at_ref, shiftT_ref, w2bt_ref, out_ref):
    ids = ids_ref[0]
    oh = (ids == lax.broadcasted_iota(jnp.int32, (VOCAB, B), 0)
          ).astype(jnp.bfloat16)
    g = jnp.dot(at_ref[...], oh,
                preferred_element_type=jnp.float32)
    h = jnp.maximum(g + shiftT_ref[...], 0.0).astype(jnp.bfloat16)
    h_aug = jnp.concatenate([h, jnp.ones((1, B), jnp.bfloat16)], axis=0)
    out_ref[0] = jnp.dot(w2bt_ref[...], h_aug,
                         preferred_element_type=jnp.float32)


def kernel(x_chars, x_shifts, char_embed, shift_embed, W1, b1, W2, b2):
    x_chars = x_chars.astype(jnp.int32)
    x_shifts = x_shifts.astype(jnp.int32)
    w1b = jnp.concatenate([W1, b1[None, :]], axis=0)
    w2bt = jnp.concatenate([W2.T, b2[:, None]],
                           axis=1).astype(jnp.bfloat16)

    at_tab, shiftT = pl.pallas_call(
        _precompute_body,
        out_shape=(
            jax.ShapeDtypeStruct((D, VOCAB), jnp.bfloat16),
            jax.ShapeDtypeStruct((D, B), jnp.float32),
        ),
    )(w1b, char_embed, shift_embed, x_shifts.reshape(1, B))

    ids_t = x_chars.T.reshape(S, 1, B)

    res = pl.pallas_call(
        _main_body,
        grid=(S,),
        in_specs=[
            pl.BlockSpec((1, 1, B), lambda s: (s, 0, 0)),
            pl.BlockSpec((D, VOCAB), lambda s: (0, 0)),
            pl.BlockSpec((D, B), lambda s: (0, 0)),
            pl.BlockSpec((VOCAB, D + 1), lambda s: (0, 0)),
        ],
        out_specs=pl.BlockSpec((1, VOCAB, B), lambda s: (s, 0, 0)),
        out_shape=jax.ShapeDtypeStruct((S, VOCAB, B), jnp.float32),
    )(ids_t, at_tab, shiftT, w2bt)

    return jnp.transpose(res, (2, 0, 1))


# speedup vs baseline: 4.6484x; 1.0524x over previous
import jax
import jax.numpy as jnp
from jax import lax
from jax.experimental import pallas as pl

VOCAB = 1000
D = 128
B, S = 1024, 20
NSHIFT = 26


def _precompute_body(w1_ref, b1_ref, char_ref, shift_ref, shifts_ref,
                     w2_ref, b2_ref, at_ref, shiftT_ref, w2b_ref):
    w1c = w1_ref[0:D]
    at_ref[...] = lax.dot_general(
        w1c, char_ref[...], (((0,), (1,)), ((), ())),
        preferred_element_type=jnp.float32).astype(jnp.bfloat16)
    w1s_aug = jnp.concatenate([w1_ref[D:], b1_ref[...]], axis=0)
    se_aug = jnp.concatenate(
        [shift_ref[...], jnp.ones((NSHIFT, 1), jnp.float32)], axis=1)
    ct = lax.dot_general(
        w1s_aug, se_aug, (((0,), (1,)), ((), ())),
        preferred_element_type=jnp.float32)
    ohs = (shifts_ref[...] ==
           lax.broadcasted_iota(jnp.int32, (NSHIFT, B), 0)
           ).astype(jnp.float32)
    shiftT_ref[...] = jnp.dot(ct, ohs, preferred_element_type=jnp.float32)
    w2b_ref[...] = jnp.concatenate(
        [w2_ref[...], b2_ref[...]], axis=0).astype(jnp.bfloat16)


def _main_body(ids_ref, at_ref, shiftT_ref, w2b_ref, out_ref):
    ids = ids_ref[0]
    oh = (ids == lax.broadcasted_iota(jnp.int32, (VOCAB, B), 0)
          ).astype(jnp.bfloat16)
    g = jnp.dot(at_ref[...], oh,
                preferred_element_type=jnp.float32)
    h = jnp.maximum(g + shiftT_ref[...], 0.0).astype(jnp.bfloat16)
    h_aug = jnp.concatenate([h, jnp.ones((1, B), jnp.bfloat16)], axis=0)
    out_ref[0] = lax.dot_general(
        w2b_ref[...], h_aug, (((0,), (0,)), ((), ())),
        preferred_element_type=jnp.float32)


def kernel(x_chars, x_shifts, char_embed, shift_embed, W1, b1, W2, b2):
    x_chars = x_chars.astype(jnp.int32)
    x_shifts = x_shifts.astype(jnp.int32)

    at_tab, shiftT, w2b = pl.pallas_call(
        _precompute_body,
        out_shape=(
            jax.ShapeDtypeStruct((D, VOCAB), jnp.bfloat16),
            jax.ShapeDtypeStruct((D, B), jnp.float32),
            jax.ShapeDtypeStruct((D + 1, VOCAB), jnp.bfloat16),
        ),
    )(W1, b1.reshape(1, D), char_embed, shift_embed, x_shifts.reshape(1, B),
      W2, b2.reshape(1, VOCAB))

    res = pl.pallas_call(
        _main_body,
        grid=(S,),
        in_specs=[
            pl.BlockSpec((1, 1, B), lambda s: (s, 0, 0)),
            pl.BlockSpec((D, VOCAB), lambda s: (0, 0)),
            pl.BlockSpec((D, B), lambda s: (0, 0)),
            pl.BlockSpec((D + 1, VOCAB), lambda s: (0, 0)),
        ],
        out_specs=pl.BlockSpec((1, VOCAB, B), lambda s: (s, 0, 0)),
        out_shape=jax.ShapeDtypeStruct((S, VOCAB, B), jnp.float32),
    )(x_chars.T.reshape(S, 1, B), at_tab, shiftT, w2b)

    return jnp.transpose(res, (2, 0, 1))


# speedup vs baseline: 4.8998x; 1.0541x over previous
import jax
import jax.numpy as jnp
from jax import lax
from jax.experimental import pallas as pl
from jax.experimental.pallas import tpu as pltpu

VOCAB = 1000
D = 128
B, S = 1024, 20
NSHIFT = 26


def _body(ids_ref, w1_ref, b1_ref, char_ref, shift_ref, shifts_ref,
          w2_ref, b2_ref, out_ref, at_s, shiftT_s, w2b_s):
    @pl.when(pl.program_id(0) == 0)
    def _precompute():
        w1c = w1_ref[0:D]
        at_s[...] = lax.dot_general(
            w1c, char_ref[...], (((0,), (1,)), ((), ())),
            preferred_element_type=jnp.float32).astype(jnp.bfloat16)
        w1s_aug = jnp.concatenate([w1_ref[D:], b1_ref[...]], axis=0)
        se_aug = jnp.concatenate(
            [shift_ref[...], jnp.ones((NSHIFT, 1), jnp.float32)], axis=1)
        ct = lax.dot_general(
            w1s_aug, se_aug, (((0,), (1,)), ((), ())),
            preferred_element_type=jnp.float32)
        ohs = (shifts_ref[...] ==
               lax.broadcasted_iota(jnp.int32, (NSHIFT, B), 0)
               ).astype(jnp.float32)
        shiftT_s[...] = jnp.dot(ct, ohs, preferred_element_type=jnp.float32)
        w2b_s[...] = jnp.concatenate(
            [w2_ref[...], b2_ref[...]], axis=0).astype(jnp.bfloat16)

    ids = ids_ref[0]
    oh = (ids == lax.broadcasted_iota(jnp.int32, (VOCAB, B), 0)
          ).astype(jnp.bfloat16)
    g = jnp.dot(at_s[...], oh,
                preferred_element_type=jnp.float32)
    h = jnp.maximum(g + shiftT_s[...], 0.0).astype(jnp.bfloat16)
    h_aug = jnp.concatenate([h, jnp.ones((1, B), jnp.bfloat16)], axis=0)
    out_ref[0] = lax.dot_general(
        w2b_s[...], h_aug, (((0,), (0,)), ((), ())),
        preferred_element_type=jnp.float32)


def kernel(x_chars, x_shifts, char_embed, shift_embed, W1, b1, W2, b2):
    x_chars = x_chars.astype(jnp.int32)
    x_shifts = x_shifts.astype(jnp.int32)

    res = pl.pallas_call(
        _body,
        grid=(S,),
        in_specs=[
            pl.BlockSpec((1, 1, B), lambda s: (s, 0, 0)),
            pl.BlockSpec((2 * D, D), lambda s: (0, 0)),
            pl.BlockSpec((1, D), lambda s: (0, 0)),
            pl.BlockSpec((VOCAB, D), lambda s: (0, 0)),
            pl.BlockSpec((NSHIFT, D), lambda s: (0, 0)),
            pl.BlockSpec((1, B), lambda s: (0, 0)),
            pl.BlockSpec((D, VOCAB), lambda s: (0, 0)),
            pl.BlockSpec((1, VOCAB), lambda s: (0, 0)),
        ],
        out_specs=pl.BlockSpec((1, VOCAB, B), lambda s: (s, 0, 0)),
        out_shape=jax.ShapeDtypeStruct((S, VOCAB, B), jnp.float32),
        scratch_shapes=[
            pltpu.VMEM((D, VOCAB), jnp.bfloat16),
            pltpu.VMEM((D, B), jnp.float32),
            pltpu.VMEM((D + 1, VOCAB), jnp.bfloat16),
        ],
    )(x_chars.T.reshape(S, 1, B), W1, b1.reshape(1, D), char_embed,
      shift_embed, x_shifts.reshape(1, B), W2, b2.reshape(1, VOCAB))

    return jnp.transpose(res, (2, 0, 1))
